# baseline (speedup 1.0000x reference)
import functools

import jax
import jax.numpy as jnp
from jax import lax
from jax.experimental import pallas as pl
from jax.experimental.pallas import tpu as pltpu
from jax.experimental.pallas import tpu_sc as plsc

_NUM_CORES = 2
_NUM_SUBCORES = 16
_NUM_WORKERS = _NUM_CORES * _NUM_SUBCORES
_GROUP = 16


def _make_scatter(M, D, B):
  per_worker = B // _NUM_WORKERS
  n_groups = per_worker // _GROUP
  assert per_worker % _GROUP == 0

  mesh = plsc.VectorSubcoreMesh(core_axis_name="c", subcore_axis_name="s")

  @functools.partial(
      pl.kernel,
      out_type=(),
      mesh=mesh,
      scratch_types=[
          pltpu.VMEM((per_worker,), jnp.int32),
          pltpu.VMEM((per_worker,), jnp.int32),
          pltpu.SemaphoreType.DMA,
      ],
  )
  def scatter_kernel(sidx_hbm, src_hbm, val_hbm, table_ref,
                     sidx_v, src_v, sem):
    wid = lax.axis_index("s") * _NUM_CORES + lax.axis_index("c")
    pltpu.sync_copy(sidx_hbm.at[pl.ds(wid * per_worker, per_worker)], sidx_v)
    pltpu.sync_copy(src_hbm.at[pl.ds(wid * per_worker, per_worker)], src_v)

    @pl.loop(0, n_groups)
    def _fire(g):
      dv = sidx_v[pl.ds(g * _GROUP, _GROUP)]
      gv = src_v[pl.ds(g * _GROUP, _GROUP)]
      for l in range(_GROUP):
        pltpu.async_copy(val_hbm.at[pl.ds(gv[l], 1), :],
                         table_ref.at[pl.ds(dv[l], 1), :], sem)

    @pl.loop(0, per_worker)
    def _drain(m):
      pltpu.make_async_copy(val_hbm.at[pl.ds(0, 1), :],
                            table_ref.at[pl.ds(0, 1), :], sem).wait()

  return scatter_kernel


def kernel(mem, idx, val):
  M, D = mem.shape
  B = idx.shape[0]

  pos = jnp.arange(B, dtype=jnp.int32)
  last = jnp.zeros((M,), jnp.int32).at[idx].max(pos, mode="drop",
                                               unique_indices=False)
  src = jnp.take(last, idx)

  table_ref = jax.new_ref(mem)
  _make_scatter(M, D, B)(idx, src, val, table_ref)
  return table_ref[...]


# speedup vs baseline: 1.0006x; 1.0006x over previous
import functools

import jax
import jax.numpy as jnp
from jax import lax
from jax.experimental import pallas as pl
from jax.experimental.pallas import tpu as pltpu
from jax.experimental.pallas import tpu_sc as plsc

_NUM_CORES = 2
_NUM_SUBCORES = 16
_NUM_WORKERS = _NUM_CORES * _NUM_SUBCORES
_GROUP = 16


def _make_scatter(M, D, B):
  per_worker = B // _NUM_WORKERS
  n_groups = per_worker // _GROUP
  assert per_worker % _GROUP == 0

  mesh = plsc.VectorSubcoreMesh(core_axis_name="c", subcore_axis_name="s")

  @functools.partial(
      pl.kernel,
      out_type=(),
      mesh=mesh,
      scratch_types=[
          pltpu.VMEM((per_worker,), jnp.int32),
          pltpu.VMEM((per_worker,), jnp.int32),
          pltpu.SemaphoreType.DMA,
          pltpu.SemaphoreType.DMA,
          pltpu.SemaphoreType.DMA,
          pltpu.SemaphoreType.DMA,
      ],
  )
  def scatter_kernel(sidx_hbm, src_hbm, val_hbm, table_ref,
                     sidx_v, src_v, *sems):
    wid = lax.axis_index("s") * _NUM_CORES + lax.axis_index("c")
    pltpu.sync_copy(sidx_hbm.at[pl.ds(wid * per_worker, per_worker)], sidx_v)
    pltpu.sync_copy(src_hbm.at[pl.ds(wid * per_worker, per_worker)], src_v)

    @pl.loop(0, n_groups)
    def _fire(g):
      dv = sidx_v[pl.ds(g * _GROUP, _GROUP)]
      gv = src_v[pl.ds(g * _GROUP, _GROUP)]
      for l in range(_GROUP):
        pltpu.async_copy(val_hbm.at[pl.ds(gv[l], 1), :],
                         table_ref.at[pl.ds(dv[l], 1), :], sems[l % 4])

    @pl.loop(0, n_groups)
    def _drain(g):
      for l in range(_GROUP):
        pltpu.make_async_copy(val_hbm.at[pl.ds(0, 1), :],
                              table_ref.at[pl.ds(0, 1), :],
                              sems[l % 4]).wait()

  return scatter_kernel


def kernel(mem, idx, val):
  M, D = mem.shape
  B = idx.shape[0]

  pos = jnp.arange(B, dtype=jnp.int32)
  last = jnp.zeros((M,), jnp.int32).at[idx].max(pos, mode="drop",
                                               unique_indices=False)
  src = jnp.take(last, idx)

  table_ref = jax.new_ref(mem)
  _make_scatter(M, D, B)(idx, src, val, table_ref)
  return table_ref[...]


# speedup vs baseline: 1.3004x; 1.2995x over previous
import functools

import jax
import jax.numpy as jnp
from jax import lax
from jax.experimental import pallas as pl
from jax.experimental.pallas import tpu as pltpu
from jax.experimental.pallas import tpu_sc as plsc

_NUM_CORES = 2
_NUM_SUBCORES = 16
_NUM_WORKERS = _NUM_CORES * _NUM_SUBCORES
_GROUP = 16


def _make_scatter(M, D, B):
  per_worker = B // _NUM_WORKERS
  n_groups = per_worker // _GROUP
  assert per_worker % _GROUP == 0

  mesh = plsc.VectorSubcoreMesh(core_axis_name="c", subcore_axis_name="s")

  @functools.partial(
      pl.kernel,
      out_type=(),
      mesh=mesh,
      scratch_types=[
          pltpu.VMEM((per_worker,), jnp.int32),
          pltpu.VMEM((per_worker,), jnp.int32),
          pltpu.VMEM((2 * _GROUP, 64), jnp.float32),
          pltpu.SemaphoreType.DMA,
          pltpu.SemaphoreType.DMA,
          pltpu.SemaphoreType.DMA,
      ],
  )
  def scatter_kernel(sidx_hbm, src_hbm, val_hbm, table_ref,
                     sidx_v, src_v, rows_v, gsem, ssem0, ssem1):
    wid = lax.axis_index("s") * _NUM_CORES + lax.axis_index("c")
    pltpu.sync_copy(sidx_hbm.at[pl.ds(wid * per_worker, per_worker)], sidx_v)
    pltpu.sync_copy(src_hbm.at[pl.ds(wid * per_worker, per_worker)], src_v)
    ssems = (ssem0, ssem1)

    @pl.loop(0, n_groups // 2)
    def _pipe(h):
      for p in range(2):
        g = h * 2 + p
        dv = sidx_v[pl.ds(g * _GROUP, _GROUP)]
        gv = src_v[pl.ds(g * _GROUP, _GROUP)]
        buf = rows_v.at[pl.ds(p * _GROUP, _GROUP)]
        for l in range(_GROUP):
          pltpu.async_copy(val_hbm.at[pl.ds(gv[l], 1), :],
                           buf.at[pl.ds(l, 1)], gsem)
        for l in range(_GROUP):
          pltpu.make_async_copy(val_hbm.at[pl.ds(0, 1), :],
                                buf.at[pl.ds(l, 1)], gsem).wait()
        for l in range(_GROUP):
          pltpu.async_copy(buf.at[pl.ds(l, 1)],
                           table_ref.at[pl.ds(dv[l], 1), :], ssems[p])
      for p in range(2):
        for l in range(_GROUP):
          pltpu.make_async_copy(rows_v.at[pl.ds(p * _GROUP + l, 1)],
                                table_ref.at[pl.ds(0, 1), :],
                                ssems[p]).wait()

  return scatter_kernel


def kernel(mem, idx, val):
  M, D = mem.shape
  B = idx.shape[0]

  pos = jnp.arange(B, dtype=jnp.int32)
  last = jnp.zeros((M,), jnp.int32).at[idx].max(pos, mode="drop",
                                               unique_indices=False)
  src = jnp.take(last, idx)

  table_ref = jax.new_ref(mem)
  _make_scatter(M, D, B)(idx, src, val, table_ref)
  return table_ref[...]
